# baseline (speedup 1.0000x reference)
import functools

import jax
import jax.numpy as jnp
from jax import lax
from jax.experimental import pallas as pl
from jax.experimental.pallas import tpu as pltpu

_D = 768
_DFF = 1536
_E = 8
_NTOK = 2048
_K = 2
_S = _NTOK * _K
_BT = 256
_BG = 256
_NB = _S // _BG + _E
_PMAX = _NB * _BG


def _gating_body(x_ref, gw_ref, gb_ref, wts_ref, idx_ref):
    x = x_ref[...]
    logits = jnp.dot(x, gw_ref[...],
                     preferred_element_type=jnp.float32) + gb_ref[...]
    iota = lax.broadcasted_iota(jnp.int32, (_BT, _E), 1)
    m1 = jnp.max(logits, axis=1, keepdims=True)
    i1 = jnp.min(jnp.where(logits >= m1, iota, _E), axis=1, keepdims=True)
    l2 = jnp.where(iota == i1, -jnp.inf, logits)
    m2 = jnp.max(l2, axis=1, keepdims=True)
    i2 = jnp.min(jnp.where(l2 >= m2, iota, _E), axis=1, keepdims=True)
    z = jnp.sum(jnp.exp(logits - m1), axis=1, keepdims=True)
    p1 = 1.0 / z
    p2 = jnp.exp(m2 - m1) / z
    t = jnp.exp(p2 - p1)
    w1 = 1.0 / (1.0 + t)
    w2 = t / (1.0 + t)
    wts_ref[...] = jnp.concatenate([w1, w2], axis=1)
    idx_ref[...] = jnp.concatenate([i1, i2], axis=1)


def _gating(x, gate_w, gate_b):
    return pl.pallas_call(
        _gating_body,
        grid=(_NTOK // _BT,),
        in_specs=[
            pl.BlockSpec((_BT, _D), lambda n: (n, 0)),
            pl.BlockSpec((_D, _E), lambda n: (0, 0)),
            pl.BlockSpec((1, _E), lambda n: (0, 0)),
        ],
        out_specs=[
            pl.BlockSpec((_BT, _K), lambda n: (n, 0)),
            pl.BlockSpec((_BT, _K), lambda n: (n, 0)),
        ],
        out_shape=[
            jax.ShapeDtypeStruct((_NTOK, _K), jnp.float32),
            jax.ShapeDtypeStruct((_NTOK, _K), jnp.int32),
        ],
        compiler_params=pltpu.CompilerParams(
            dimension_semantics=("parallel",),
        ),
    )(x, gate_w, gate_b.reshape(1, _E))


def _routing(idx, wts):
    eflat = idx.reshape(_S)
    wflat = wts.reshape(_S)
    oh = (eflat[:, None] == jnp.arange(_E)[None, :]).astype(jnp.int32)
    counts = jnp.sum(oh, axis=0)
    padded = ((counts + _BG - 1) // _BG) * _BG
    starts = jnp.concatenate([jnp.zeros((1,), jnp.int32),
                              jnp.cumsum(padded)[:-1].astype(jnp.int32)])
    rank = jnp.sum(jnp.where(oh == 1, jnp.cumsum(oh, axis=0) - 1, 0), axis=1)
    pos_flat = starts[eflat] + rank
    tok = jnp.arange(_S, dtype=jnp.int32) // _K
    gather_tok = jnp.zeros((_PMAX,), jnp.int32).at[pos_flat].set(tok)
    wslot = jnp.zeros((_PMAX,), jnp.float32).at[pos_flat].set(wflat)
    ends = starts + padded
    bstart = jnp.arange(_NB, dtype=jnp.int32) * _BG
    block_expert = jnp.sum((bstart[:, None] >= ends[None, :]).astype(jnp.int32),
                           axis=1)
    block_expert = jnp.minimum(block_expert, _E - 1)
    return gather_tok, wslot, block_expert, pos_flat.reshape(_NTOK, _K)


def _ffn_body(be_ref, x_ref, w_ref, W1_ref, b1_ref, W2_ref, b2_ref, y_ref):
    x = x_ref[...]
    h = jnp.maximum(jnp.dot(x, W1_ref[0], preferred_element_type=jnp.float32)
                    + b1_ref[0], 0.0)
    y = jnp.dot(h, W2_ref[0], preferred_element_type=jnp.float32) + b2_ref[0]
    y_ref[...] = y * w_ref[...]


def _ffn(x_g, wslot, block_expert, W1, b1, W2, b2):
    grid_spec = pltpu.PrefetchScalarGridSpec(
        num_scalar_prefetch=1,
        grid=(_NB,),
        in_specs=[
            pl.BlockSpec((_BG, _D), lambda b, be: (b, 0)),
            pl.BlockSpec((_BG, 1), lambda b, be: (b, 0)),
            pl.BlockSpec((1, _D, _DFF), lambda b, be: (be[b], 0, 0)),
            pl.BlockSpec((1, 1, _DFF), lambda b, be: (be[b], 0, 0)),
            pl.BlockSpec((1, _DFF, _D), lambda b, be: (be[b], 0, 0)),
            pl.BlockSpec((1, 1, _D), lambda b, be: (be[b], 0, 0)),
        ],
        out_specs=pl.BlockSpec((_BG, _D), lambda b, be: (b, 0)),
    )
    return pl.pallas_call(
        _ffn_body,
        grid_spec=grid_spec,
        out_shape=jax.ShapeDtypeStruct((_PMAX, _D), jnp.float32),
        compiler_params=pltpu.CompilerParams(
            dimension_semantics=("arbitrary",),
        ),
    )(block_expert, x_g, wslot.reshape(_PMAX, 1),
      W1, b1.reshape(_E, 1, _DFF), W2, b2.reshape(_E, 1, _D))


def kernel(x, gate_w, gate_b, W1, b1, W2, b2):
    wts, idx = _gating(x, gate_w, gate_b)
    gather_tok, wslot, block_expert, pos = _routing(idx, wts)
    x_g = jnp.take(x, gather_tok, axis=0, mode="clip")
    y_w = _ffn(x_g, wslot, block_expert, W1, b1, W2, b2)
    out = (jnp.take(y_w, pos[:, 0], axis=0, mode="clip")
           + jnp.take(y_w, pos[:, 1], axis=0, mode="clip"))
    return out


# speedup vs baseline: 1.0019x; 1.0019x over previous
import functools

import jax
import jax.numpy as jnp
from jax import lax
from jax.experimental import pallas as pl
from jax.experimental.pallas import tpu as pltpu

_D = 768
_DFF = 1536
_E = 8
_NTOK = 2048
_K = 2
_S = _NTOK * _K
_BT = 256
_BG = 256
_NB = _S // _BG + _E
_PMAX = _NB * _BG


def _gating_body(x_ref, gw_ref, gb_ref, wts_ref, idx_ref):
    x = x_ref[...]
    logits = jnp.dot(x, gw_ref[...],
                     preferred_element_type=jnp.float32) + gb_ref[...]
    iota = lax.broadcasted_iota(jnp.int32, (_BT, _E), 1)
    m1 = jnp.max(logits, axis=1, keepdims=True)
    i1 = jnp.min(jnp.where(logits >= m1, iota, _E), axis=1, keepdims=True)
    l2 = jnp.where(iota == i1, -jnp.inf, logits)
    m2 = jnp.max(l2, axis=1, keepdims=True)
    i2 = jnp.min(jnp.where(l2 >= m2, iota, _E), axis=1, keepdims=True)
    z = jnp.sum(jnp.exp(logits - m1), axis=1, keepdims=True)
    p1 = 1.0 / z
    p2 = jnp.exp(m2 - m1) / z
    t = jnp.exp(p2 - p1)
    w1 = 1.0 / (1.0 + t)
    w2 = t / (1.0 + t)
    wts_ref[...] = jnp.concatenate([w1, w2], axis=1)
    idx_ref[...] = jnp.concatenate([i1, i2], axis=1)


def _gating(x, gate_w, gate_b):
    return pl.pallas_call(
        _gating_body,
        grid=(_NTOK // _BT,),
        in_specs=[
            pl.BlockSpec((_BT, _D), lambda n: (n, 0)),
            pl.BlockSpec((_D, _E), lambda n: (0, 0)),
            pl.BlockSpec((1, _E), lambda n: (0, 0)),
        ],
        out_specs=[
            pl.BlockSpec((_BT, _K), lambda n: (n, 0)),
            pl.BlockSpec((_BT, _K), lambda n: (n, 0)),
        ],
        out_shape=[
            jax.ShapeDtypeStruct((_NTOK, _K), jnp.float32),
            jax.ShapeDtypeStruct((_NTOK, _K), jnp.int32),
        ],
        compiler_params=pltpu.CompilerParams(
            dimension_semantics=("parallel",),
        ),
    )(x, gate_w, gate_b.reshape(1, _E))


def _routing(idx, wts):
    eflat = idx.reshape(_S)
    wflat = wts.reshape(_S)
    oh = (eflat[:, None] == jnp.arange(_E)[None, :]).astype(jnp.int32)
    counts = jnp.sum(oh, axis=0)
    padded = ((counts + _BG - 1) // _BG) * _BG
    starts = jnp.concatenate([jnp.zeros((1,), jnp.int32),
                              jnp.cumsum(padded)[:-1].astype(jnp.int32)])
    rank = jnp.sum(jnp.where(oh == 1, jnp.cumsum(oh, axis=0) - 1, 0), axis=1)
    pos_flat = starts[eflat] + rank
    tok = jnp.arange(_S, dtype=jnp.int32) // _K
    gather_tok = jnp.zeros((_PMAX,), jnp.int32).at[pos_flat].set(tok)
    wslot = jnp.zeros((_PMAX,), jnp.float32).at[pos_flat].set(wflat)
    ends = starts + padded
    bstart = jnp.arange(_NB, dtype=jnp.int32) * _BG
    block_expert = jnp.sum((bstart[:, None] >= ends[None, :]).astype(jnp.int32),
                           axis=1)
    block_expert = jnp.minimum(block_expert, _E - 1)
    return gather_tok, wslot, block_expert, pos_flat.reshape(_NTOK, _K)


def _ffn_body(be_ref, x_ref, w_ref, W1_ref, b1_ref, W2_ref, b2_ref, y_ref):
    x = x_ref[...].astype(jnp.bfloat16)
    h = jnp.maximum(jnp.dot(x, W1_ref[0].astype(jnp.bfloat16),
                            preferred_element_type=jnp.float32) + b1_ref[0], 0.0)
    y = jnp.dot(h.astype(jnp.bfloat16), W2_ref[0].astype(jnp.bfloat16),
                preferred_element_type=jnp.float32) + b2_ref[0]
    y_ref[...] = y * w_ref[...]


def _ffn(x_g, wslot, block_expert, W1, b1, W2, b2):
    grid_spec = pltpu.PrefetchScalarGridSpec(
        num_scalar_prefetch=1,
        grid=(_NB,),
        in_specs=[
            pl.BlockSpec((_BG, _D), lambda b, be: (b, 0)),
            pl.BlockSpec((_BG, 1), lambda b, be: (b, 0)),
            pl.BlockSpec((1, _D, _DFF), lambda b, be: (be[b], 0, 0)),
            pl.BlockSpec((1, 1, _DFF), lambda b, be: (be[b], 0, 0)),
            pl.BlockSpec((1, _DFF, _D), lambda b, be: (be[b], 0, 0)),
            pl.BlockSpec((1, 1, _D), lambda b, be: (be[b], 0, 0)),
        ],
        out_specs=pl.BlockSpec((_BG, _D), lambda b, be: (b, 0)),
    )
    return pl.pallas_call(
        _ffn_body,
        grid_spec=grid_spec,
        out_shape=jax.ShapeDtypeStruct((_PMAX, _D), jnp.float32),
        compiler_params=pltpu.CompilerParams(
            dimension_semantics=("arbitrary",),
        ),
    )(block_expert, x_g, wslot.reshape(_PMAX, 1),
      W1, b1.reshape(_E, 1, _DFF), W2, b2.reshape(_E, 1, _D))


def kernel(x, gate_w, gate_b, W1, b1, W2, b2):
    wts, idx = _gating(x, gate_w, gate_b)
    gather_tok, wslot, block_expert, pos = _routing(idx, wts)
    x_g = jnp.take(x, gather_tok, axis=0, mode="clip")
    y_w = _ffn(x_g, wslot, block_expert, W1, b1, W2, b2)
    out = (jnp.take(y_w, pos[:, 0], axis=0, mode="clip")
           + jnp.take(y_w, pos[:, 1], axis=0, mode="clip"))
    return out
